# Initial kernel scaffold; baseline (speedup 1.0000x reference)
import jax
import jax.numpy as jnp
from jax.experimental import pallas as pl


def kernel(x, codebook):
    raise NotImplementedError("write your pallas kernel here")


# speedup vs baseline: 2.4935x; 2.4935x over previous
import jax
import jax.numpy as jnp
from jax.experimental import pallas as pl


def _sum_sq_tree_rows(sq):
    t = ((sq[0:8] + sq[8:16]) + sq[16:24]) + sq[24:32]
    t = t[0:4] + t[4:8]
    t = t[0:2] + t[2:4]
    return t[0:1] + t[1:2]


def _sum_sq_tree_lanes(sq):
    t = ((sq[:, 0:8] + sq[:, 8:16]) + sq[:, 16:24]) + sq[:, 24:32]
    t = t[:, 0:4] + t[:, 4:8]
    t = t[:, 0:2] + t[:, 2:4]
    return t[:, 0:1] + t[:, 1:2]


def _vq_kernel(x_ref, w_ref, zq_ref, z_ref, idx_ref):
    xb = x_ref[0]
    w = w_ref[...]
    k = w.shape[0]
    x2 = _sum_sq_tree_rows(xb * xb)
    w2 = _sum_sq_tree_lanes(w * w)
    prod = jax.lax.dot_general(
        w, xb, (((1,), (0,)), ((), ())),
        preferred_element_type=jnp.float32)
    d2 = x2 + w2 - 2.0 * prod
    dist = jnp.sqrt(jnp.clip(d2, 0.0, None))
    vmin = jnp.min(dist, axis=0, keepdims=True)
    iota = jax.lax.broadcasted_iota(jnp.int32, dist.shape, 0)
    eq = dist == vmin
    idx = jnp.min(jnp.where(eq, iota, k), axis=0)
    idx_ref[0, 0, :] = idx
    onehot = (iota == idx[None, :]).astype(jnp.float32)
    z = jax.lax.dot_general(
        w, onehot, (((0,), (0,)), ((), ())),
        preferred_element_type=jnp.float32)
    z_ref[0] = z
    zq_ref[0] = xb + (z - xb)


def kernel(x, codebook):
    b, c, l = x.shape
    k, _ = codebook.shape
    zq, z, idx3 = pl.pallas_call(
        _vq_kernel,
        grid=(b,),
        in_specs=[
            pl.BlockSpec((1, c, l), lambda i: (i, 0, 0)),
            pl.BlockSpec((k, c), lambda i: (0, 0)),
        ],
        out_specs=[
            pl.BlockSpec((1, c, l), lambda i: (i, 0, 0)),
            pl.BlockSpec((1, c, l), lambda i: (i, 0, 0)),
            pl.BlockSpec((1, 1, l), lambda i: (i, 0, 0)),
        ],
        out_shape=[
            jax.ShapeDtypeStruct((b, c, l), x.dtype),
            jax.ShapeDtypeStruct((b, c, l), x.dtype),
            jax.ShapeDtypeStruct((b, 1, l), jnp.int32),
        ],
    )(x, codebook)
    return zq, z, x, idx3.reshape(b, l)


# speedup vs baseline: 2.9113x; 1.1676x over previous
import jax
import jax.numpy as jnp
from jax.experimental import pallas as pl

_RB = 2


def _sum_sq_tree_rows(sq):
    t = ((sq[0:8] + sq[8:16]) + sq[16:24]) + sq[24:32]
    t = t[0:4] + t[4:8]
    t = t[0:2] + t[2:4]
    return t[0:1] + t[1:2]


def _sum_sq_tree_lanes(sq):
    t = ((sq[:, 0:8] + sq[:, 8:16]) + sq[:, 16:24]) + sq[:, 24:32]
    t = t[:, 0:4] + t[:, 4:8]
    t = t[:, 0:2] + t[:, 2:4]
    return t[:, 0:1] + t[:, 1:2]


def _vq_kernel(x_ref, w_ref, zq_ref, z_ref, idx_ref):
    w = w_ref[...]
    k = w.shape[0]
    wneg2 = w * (-2.0)
    w2 = _sum_sq_tree_lanes(w * w)
    for r in range(_RB):
        xb = x_ref[r]
        x2 = _sum_sq_tree_rows(xb * xb)
        prodneg = jax.lax.dot_general(
            wneg2, xb, (((1,), (0,)), ((), ())),
            preferred_element_type=jnp.float32)
        dist = jnp.sqrt(jnp.clip((x2 + w2) + prodneg, 0.0, None))
        vmin = jnp.min(dist, axis=0, keepdims=True)
        iota = jax.lax.broadcasted_iota(jnp.int32, dist.shape, 0)
        idx = jnp.min(jnp.where(dist == vmin, iota, k), axis=0)
        idx_ref[0, r, :] = idx
        onehot = (iota == idx[None, :]).astype(jnp.float32)
        z = jax.lax.dot_general(
            w, onehot, (((0,), (0,)), ((), ())),
            preferred_element_type=jnp.float32)
        z_ref[r] = z
        zq_ref[r] = xb + (z - xb)


def kernel(x, codebook):
    b, c, l = x.shape
    k, _ = codebook.shape
    zq, z, idx3 = pl.pallas_call(
        _vq_kernel,
        grid=(b // _RB,),
        in_specs=[
            pl.BlockSpec((_RB, c, l), lambda i: (i, 0, 0)),
            pl.BlockSpec((k, c), lambda i: (0, 0)),
        ],
        out_specs=[
            pl.BlockSpec((_RB, c, l), lambda i: (i, 0, 0)),
            pl.BlockSpec((_RB, c, l), lambda i: (i, 0, 0)),
            pl.BlockSpec((1, _RB, l), lambda i: (i, 0, 0)),
        ],
        out_shape=[
            jax.ShapeDtypeStruct((b, c, l), x.dtype),
            jax.ShapeDtypeStruct((b, c, l), x.dtype),
            jax.ShapeDtypeStruct((b // _RB, _RB, l), jnp.int32),
        ],
    )(x, codebook)
    return zq, z, x, idx3.reshape(b, l)


# speedup vs baseline: 3.0825x; 1.0588x over previous
import jax
import jax.numpy as jnp
from jax.experimental import pallas as pl

_RB = 4


def _sum_sq_tree_rows(sq):
    t = ((sq[0:8] + sq[8:16]) + sq[16:24]) + sq[24:32]
    t = t[0:4] + t[4:8]
    t = t[0:2] + t[2:4]
    return t[0:1] + t[1:2]


def _sum_sq_tree_lanes(sq):
    t = ((sq[:, 0:8] + sq[:, 8:16]) + sq[:, 16:24]) + sq[:, 24:32]
    t = t[:, 0:4] + t[:, 4:8]
    t = t[:, 0:2] + t[:, 2:4]
    return t[:, 0:1] + t[:, 1:2]


def _vq_kernel(x_ref, w_ref, zq_ref, z_ref, idx_ref):
    w = w_ref[...]
    k = w.shape[0]
    wneg2 = w * (-2.0)
    w2 = _sum_sq_tree_lanes(w * w)
    for r in range(_RB):
        xb = x_ref[r]
        x2 = _sum_sq_tree_rows(xb * xb)
        prodneg = jax.lax.dot_general(
            wneg2, xb, (((1,), (0,)), ((), ())),
            preferred_element_type=jnp.float32)
        dist = jnp.sqrt(jnp.clip((x2 + w2) + prodneg, 0.0, None))
        vmin = jnp.min(dist, axis=0, keepdims=True)
        iota = jax.lax.broadcasted_iota(jnp.int32, dist.shape, 0)
        idx = jnp.min(jnp.where(dist == vmin, iota, k), axis=0)
        idx_ref[0, r, :] = idx
        onehot = (iota == idx[None, :]).astype(jnp.float32)
        z = jax.lax.dot_general(
            w, onehot, (((0,), (0,)), ((), ())),
            preferred_element_type=jnp.float32)
        z_ref[r] = z
        zq_ref[r] = xb + (z - xb)


def kernel(x, codebook):
    b, c, l = x.shape
    k, _ = codebook.shape
    zq, z, idx3 = pl.pallas_call(
        _vq_kernel,
        grid=(b // _RB,),
        in_specs=[
            pl.BlockSpec((_RB, c, l), lambda i: (i, 0, 0)),
            pl.BlockSpec((k, c), lambda i: (0, 0)),
        ],
        out_specs=[
            pl.BlockSpec((_RB, c, l), lambda i: (i, 0, 0)),
            pl.BlockSpec((_RB, c, l), lambda i: (i, 0, 0)),
            pl.BlockSpec((1, _RB, l), lambda i: (i, 0, 0)),
        ],
        out_shape=[
            jax.ShapeDtypeStruct((b, c, l), x.dtype),
            jax.ShapeDtypeStruct((b, c, l), x.dtype),
            jax.ShapeDtypeStruct((b // _RB, _RB, l), jnp.int32),
        ],
    )(x, codebook)
    return zq, z, x, idx3.reshape(b, l)


# speedup vs baseline: 3.1652x; 1.0268x over previous
import jax
import jax.numpy as jnp
from jax.experimental import pallas as pl

_RB = 4


def _sum_sq_tree_rows(sq):
    t = ((sq[0:8] + sq[8:16]) + sq[16:24]) + sq[24:32]
    t = t[0:4] + t[4:8]
    t = t[0:2] + t[2:4]
    return t[0:1] + t[1:2]


def _sum_sq_tree_lanes(sq):
    t = ((sq[:, 0:8] + sq[:, 8:16]) + sq[:, 16:24]) + sq[:, 24:32]
    t = t[:, 0:4] + t[:, 4:8]
    t = t[:, 0:2] + t[:, 2:4]
    return t[:, 0:1] + t[:, 1:2]


def _vq_kernel(x_ref, w_ref, zq_ref, z_ref, idx_ref):
    w = w_ref[...]
    k = w.shape[0]
    wneg2 = w * (-2.0)
    w2 = _sum_sq_tree_lanes(w * w)
    for r in range(_RB):
        xb = x_ref[r]
        x2 = _sum_sq_tree_rows(xb * xb)
        prodneg = jax.lax.dot_general(
            wneg2, xb, (((1,), (0,)), ((), ())),
            preferred_element_type=jnp.float32)
        dist = jnp.sqrt(jnp.clip((x2 + w2) + prodneg, 0.0, None))
        vmin = jnp.min(dist, axis=0, keepdims=True)
        iotaf = jax.lax.broadcasted_iota(
            jnp.int32, (k, 1), 0).astype(jnp.float32)
        idxf = jnp.min(jnp.where(dist == vmin, iotaf, float(k)),
                       axis=0, keepdims=True)
        idx_ref[0, r, :] = idxf.astype(jnp.int32)[0]
        onehot = (iotaf == idxf).astype(jnp.float32)
        z = jax.lax.dot_general(
            w, onehot, (((0,), (0,)), ((), ())),
            preferred_element_type=jnp.float32)
        z_ref[r] = z
        zq_ref[r] = xb + (z - xb)


def kernel(x, codebook):
    b, c, l = x.shape
    k, _ = codebook.shape
    zq, z, idx3 = pl.pallas_call(
        _vq_kernel,
        grid=(b // _RB,),
        in_specs=[
            pl.BlockSpec((_RB, c, l), lambda i: (i, 0, 0)),
            pl.BlockSpec((k, c), lambda i: (0, 0)),
        ],
        out_specs=[
            pl.BlockSpec((_RB, c, l), lambda i: (i, 0, 0)),
            pl.BlockSpec((_RB, c, l), lambda i: (i, 0, 0)),
            pl.BlockSpec((1, _RB, l), lambda i: (i, 0, 0)),
        ],
        out_shape=[
            jax.ShapeDtypeStruct((b, c, l), x.dtype),
            jax.ShapeDtypeStruct((b, c, l), x.dtype),
            jax.ShapeDtypeStruct((b // _RB, _RB, l), jnp.int32),
        ],
    )(x, codebook)
    return zq, z, x, idx3.reshape(b, l)


# speedup vs baseline: 3.3453x; 1.0569x over previous
import jax
import jax.numpy as jnp
from jax.experimental import pallas as pl

_RB = 8


def _sum_sq_tree_rows(sq):
    t = ((sq[0:8] + sq[8:16]) + sq[16:24]) + sq[24:32]
    t = t[0:4] + t[4:8]
    t = t[0:2] + t[2:4]
    return t[0:1] + t[1:2]


def _sum_sq_tree_lanes(sq):
    t = ((sq[:, 0:8] + sq[:, 8:16]) + sq[:, 16:24]) + sq[:, 24:32]
    t = t[:, 0:4] + t[:, 4:8]
    t = t[:, 0:2] + t[:, 2:4]
    return t[:, 0:1] + t[:, 1:2]


def _vq_kernel(x_ref, w_ref, zq_ref, z_ref, idx_ref):
    w = w_ref[...]
    k = w.shape[0]
    wneg2 = w * (-2.0)
    w2 = _sum_sq_tree_lanes(w * w)
    for r in range(_RB):
        xb = x_ref[r]
        x2 = _sum_sq_tree_rows(xb * xb)
        prodneg = jax.lax.dot_general(
            wneg2, xb, (((1,), (0,)), ((), ())),
            preferred_element_type=jnp.float32)
        dist = jnp.sqrt(jnp.clip((x2 + w2) + prodneg, 0.0, None))
        vmin = jnp.min(dist, axis=0, keepdims=True)
        iotaf = jax.lax.broadcasted_iota(
            jnp.int32, (k, 1), 0).astype(jnp.float32)
        idxf = jnp.min(jnp.where(dist == vmin, iotaf, float(k)),
                       axis=0, keepdims=True)
        idx_ref[0, r, :] = idxf.astype(jnp.int32)[0]
        onehot = (iotaf == idxf).astype(jnp.float32)
        z = jax.lax.dot_general(
            w, onehot, (((0,), (0,)), ((), ())),
            preferred_element_type=jnp.float32)
        z_ref[r] = z
        zq_ref[r] = xb + (z - xb)


def kernel(x, codebook):
    b, c, l = x.shape
    k, _ = codebook.shape
    zq, z, idx3 = pl.pallas_call(
        _vq_kernel,
        grid=(b // _RB,),
        in_specs=[
            pl.BlockSpec((_RB, c, l), lambda i: (i, 0, 0)),
            pl.BlockSpec((k, c), lambda i: (0, 0)),
        ],
        out_specs=[
            pl.BlockSpec((_RB, c, l), lambda i: (i, 0, 0)),
            pl.BlockSpec((_RB, c, l), lambda i: (i, 0, 0)),
            pl.BlockSpec((1, _RB, l), lambda i: (i, 0, 0)),
        ],
        out_shape=[
            jax.ShapeDtypeStruct((b, c, l), x.dtype),
            jax.ShapeDtypeStruct((b, c, l), x.dtype),
            jax.ShapeDtypeStruct((b // _RB, _RB, l), jnp.int32),
        ],
    )(x, codebook)
    return zq, z, x, idx3.reshape(b, l)


# speedup vs baseline: 3.4413x; 1.0287x over previous
import jax
import jax.numpy as jnp
from jax.experimental import pallas as pl

_RB = 16


def _sum_sq_tree_rows(sq):
    t = ((sq[0:8] + sq[8:16]) + sq[16:24]) + sq[24:32]
    t = t[0:4] + t[4:8]
    t = t[0:2] + t[2:4]
    return t[0:1] + t[1:2]


def _sum_sq_tree_lanes(sq):
    t = ((sq[:, 0:8] + sq[:, 8:16]) + sq[:, 16:24]) + sq[:, 24:32]
    t = t[:, 0:4] + t[:, 4:8]
    t = t[:, 0:2] + t[:, 2:4]
    return t[:, 0:1] + t[:, 1:2]


def _vq_kernel(x_ref, w_ref, zq_ref, z_ref, xo_ref, idx_ref):
    w = w_ref[...]
    k = w.shape[0]
    wneg2 = w * (-2.0)
    w2 = _sum_sq_tree_lanes(w * w)
    for r in range(_RB):
        xb = x_ref[r]
        x2 = _sum_sq_tree_rows(xb * xb)
        prodneg = jax.lax.dot_general(
            wneg2, xb, (((1,), (0,)), ((), ())),
            preferred_element_type=jnp.float32)
        dist = jnp.sqrt(jnp.clip((x2 + w2) + prodneg, 0.0, None))
        vmin = jnp.min(dist, axis=0, keepdims=True)
        iotaf = jax.lax.broadcasted_iota(
            jnp.int32, (k, 1), 0).astype(jnp.float32)
        idxf = jnp.min(jnp.where(dist == vmin, iotaf, float(k)),
                       axis=0, keepdims=True)
        idx_ref[0, r, :] = idxf.astype(jnp.int32)[0]
        onehot = (iotaf == idxf).astype(jnp.float32)
        z = jax.lax.dot_general(
            w, onehot, (((0,), (0,)), ((), ())),
            preferred_element_type=jnp.float32)
        z_ref[r] = z
        zq_ref[r] = xb + (z - xb)
        xo_ref[r] = xb


def kernel(x, codebook):
    b, c, l = x.shape
    k, _ = codebook.shape
    zq, z, xo, idx3 = pl.pallas_call(
        _vq_kernel,
        grid=(b // _RB,),
        in_specs=[
            pl.BlockSpec((_RB, c, l), lambda i: (i, 0, 0)),
            pl.BlockSpec((k, c), lambda i: (0, 0)),
        ],
        out_specs=[
            pl.BlockSpec((_RB, c, l), lambda i: (i, 0, 0)),
            pl.BlockSpec((_RB, c, l), lambda i: (i, 0, 0)),
            pl.BlockSpec((_RB, c, l), lambda i: (i, 0, 0)),
            pl.BlockSpec((1, _RB, l), lambda i: (i, 0, 0)),
        ],
        out_shape=[
            jax.ShapeDtypeStruct((b, c, l), x.dtype),
            jax.ShapeDtypeStruct((b, c, l), x.dtype),
            jax.ShapeDtypeStruct((b, c, l), x.dtype),
            jax.ShapeDtypeStruct((b // _RB, _RB, l), jnp.int32),
        ],
    )(x, codebook)
    return zq, z, xo, idx3.reshape(b, l)


# speedup vs baseline: 3.4569x; 1.0045x over previous
import jax
import jax.numpy as jnp
from jax.experimental import pallas as pl

_RB = 8


def _sum_sq_tree_rows(sq):
    t = ((sq[0:8] + sq[8:16]) + sq[16:24]) + sq[24:32]
    t = t[0:4] + t[4:8]
    t = t[0:2] + t[2:4]
    return t[0:1] + t[1:2]


def _sum_sq_tree_lanes(sq):
    t = ((sq[:, 0:8] + sq[:, 8:16]) + sq[:, 16:24]) + sq[:, 24:32]
    t = t[:, 0:4] + t[:, 4:8]
    t = t[:, 0:2] + t[:, 2:4]
    return t[:, 0:1] + t[:, 1:2]


def _vq_kernel(x_ref, w_ref, zq_ref, z_ref, xo_ref, idx_ref):
    w = w_ref[...]
    k = w.shape[0]
    wneg2 = w * (-2.0)
    w2 = _sum_sq_tree_lanes(w * w)
    for r in range(_RB):
        xb = x_ref[r]
        x2 = _sum_sq_tree_rows(xb * xb)
        prodneg = jax.lax.dot_general(
            wneg2, xb, (((1,), (0,)), ((), ())),
            preferred_element_type=jnp.float32)
        dist = jnp.sqrt(jnp.clip((x2 + w2) + prodneg, 0.0, None))
        vmin = jnp.min(dist, axis=0, keepdims=True)
        iotaf = jax.lax.broadcasted_iota(
            jnp.int32, (k, 1), 0).astype(jnp.float32)
        idxf = jnp.min(jnp.where(dist == vmin, iotaf, float(k)),
                       axis=0, keepdims=True)
        idx_ref[0, r, :] = idxf.astype(jnp.int32)[0]
        onehot = (iotaf == idxf).astype(jnp.float32)
        z = jax.lax.dot_general(
            w, onehot, (((0,), (0,)), ((), ())),
            preferred_element_type=jnp.float32)
        z_ref[r] = z
        zq_ref[r] = xb + (z - xb)
        xo_ref[r] = xb


def kernel(x, codebook):
    b, c, l = x.shape
    k, _ = codebook.shape
    zq, z, xo, idx3 = pl.pallas_call(
        _vq_kernel,
        grid=(b // _RB,),
        in_specs=[
            pl.BlockSpec((_RB, c, l), lambda i: (i, 0, 0)),
            pl.BlockSpec((k, c), lambda i: (0, 0)),
        ],
        out_specs=[
            pl.BlockSpec((_RB, c, l), lambda i: (i, 0, 0)),
            pl.BlockSpec((_RB, c, l), lambda i: (i, 0, 0)),
            pl.BlockSpec((_RB, c, l), lambda i: (i, 0, 0)),
            pl.BlockSpec((1, _RB, l), lambda i: (i, 0, 0)),
        ],
        out_shape=[
            jax.ShapeDtypeStruct((b, c, l), x.dtype),
            jax.ShapeDtypeStruct((b, c, l), x.dtype),
            jax.ShapeDtypeStruct((b, c, l), x.dtype),
            jax.ShapeDtypeStruct((b // _RB, _RB, l), jnp.int32),
        ],
    )(x, codebook)
    return zq, z, xo, idx3.reshape(b, l)
